# Initial kernel scaffold; baseline (speedup 1.0000x reference)
import jax
import jax.numpy as jnp
from jax.experimental import pallas as pl


def kernel(x, edge_index, edge_attr, phm_rule, W, b):
    raise NotImplementedError("write your pallas kernel here")


# speedup vs baseline: 2.1127x; 2.1127x over previous
import functools

import jax
import jax.numpy as jnp
from jax import lax
from jax.experimental import pallas as pl
from jax.experimental.pallas import tpu as pltpu
from jax.experimental.pallas import tpu_sc as plsc

N = 10000
E = 160000
D = 256
HALF = D // 2

NUM_CORES = 2
NUM_SUBCORES = 16
EDGES_PER_TILE = E // NUM_SUBCORES
CHUNK = 80
NCHUNKS = EDGES_PER_TILE // CHUNK
ZROWS = 80
NZCHUNKS = N // ZROWS
LANES = 16


def _sc_body(x2, ea2, src_hbm, dst_hbm, out_hbm,
             agg_sh, zbuf, src_v, dst_v, eidx_v, xrow_v, earow_v,
             sem_x, sem_e):
    c = lax.axis_index("c")
    s = lax.axis_index("s")

    def _zero_row(r, _):
        for k in range(HALF // LANES):
            zbuf[r, pl.ds(k * LANES, LANES)] = jnp.zeros((LANES,), jnp.float32)
        return _
    lax.fori_loop(0, ZROWS, _zero_row, None)

    def _zero_chunk(t, _):
        ch = s + t * NUM_SUBCORES
        @pl.when(ch < NZCHUNKS)
        def _():
            pltpu.sync_copy(zbuf, agg_sh.at[pl.ds(ch * ZROWS, ZROWS)])
        return _
    lax.fori_loop(0, pl.cdiv(NZCHUNKS, NUM_SUBCORES), _zero_chunk, None)
    plsc.subcore_barrier()

    def _chunk(t, _):
        base = s * EDGES_PER_TILE + t * CHUNK
        pltpu.sync_copy(src_hbm.at[pl.ds(base, CHUNK)], src_v)
        pltpu.sync_copy(dst_hbm.at[pl.ds(base, CHUNK)], dst_v)
        iota = lax.iota(jnp.int32, LANES)
        for k in range(CHUNK // LANES):
            sl = pl.ds(k * LANES, LANES)
            src_v[sl] = src_v[sl] * 2 + c
            eidx_v[sl] = iota * 2 + (2 * (base + k * LANES) + c)
        gx = pltpu.async_copy(x2.at[src_v], xrow_v, sem_x)
        ge = pltpu.async_copy(ea2.at[eidx_v], earow_v, sem_e)
        gx.wait()
        ge.wait()

        def _add_row(r, _):
            for k in range(HALF // LANES):
                sl = pl.ds(k * LANES, LANES)
                xrow_v[r, sl] = xrow_v[r, sl] + earow_v[r, sl]
            return _
        lax.fori_loop(0, CHUNK, _add_row, None)

        pltpu.sync_copy(xrow_v, agg_sh.at[dst_v], add=True)
        return _
    lax.fori_loop(0, NCHUNKS, _chunk, None)

    plsc.subcore_barrier()

    def _wb_chunk(t, _):
        ch = s + t * NUM_SUBCORES
        @pl.when(ch < NZCHUNKS)
        def _():
            off = ch * ZROWS
            pltpu.sync_copy(agg_sh.at[pl.ds(off, ZROWS)], zbuf)
            pltpu.sync_copy(zbuf, out_hbm.at[pl.ds(c * N + off, ZROWS)])
        return _
    lax.fori_loop(0, pl.cdiv(NZCHUNKS, NUM_SUBCORES), _wb_chunk, None)


def _sc_aggregate(x2, ea2, src, dst):
    mesh = plsc.VectorSubcoreMesh(
        core_axis_name="c", subcore_axis_name="s",
        num_cores=NUM_CORES, num_subcores=NUM_SUBCORES)
    return pl.kernel(
        _sc_body,
        out_type=jax.ShapeDtypeStruct((2 * N, HALF), jnp.float32),
        mesh=mesh,
        scratch_types=[
            pltpu.VMEM_SHARED((N, HALF), jnp.float32),
            pltpu.VMEM((ZROWS, HALF), jnp.float32),
            pltpu.VMEM((CHUNK,), jnp.int32),
            pltpu.VMEM((CHUNK,), jnp.int32),
            pltpu.VMEM((CHUNK,), jnp.int32),
            pltpu.VMEM((CHUNK, HALF), jnp.float32),
            pltpu.VMEM((CHUNK, HALF), jnp.float32),
            pltpu.SemaphoreType.DMA,
            pltpu.SemaphoreType.DMA,
        ],
    )(x2, ea2, src, dst)


ROW_BLK = 1000
PHM = 4
WBLK = 64


def _tc_body(pr_ref, w_ref, a0_ref, a1_ref, x_ref, b_ref, o_ref, h_ref):
    @pl.when(pl.program_id(0) == 0)
    def _build_h():
        for i in range(PHM):
            for j in range(PHM):
                acc = pr_ref[0, i, j] * w_ref[0]
                for n in range(1, PHM):
                    acc = acc + pr_ref[n, i, j] * w_ref[n]
                h_ref[i * WBLK:(i + 1) * WBLK, j * WBLK:(j + 1) * WBLK] = acc

    h0 = h_ref[:, 0:HALF]
    h1 = h_ref[:, HALF:D]
    dims = (((1,), (1,)), ((), ()))
    out = lax.dot_general(a0_ref[...], h0, dims,
                          preferred_element_type=jnp.float32)
    out = out + lax.dot_general(a1_ref[...], h1, dims,
                                preferred_element_type=jnp.float32)
    o_ref[...] = out + x_ref[...] + b_ref[...]


def _tc_phm(phm_rule, W, a0, a1, x, b2):
    grid = (N // ROW_BLK,)
    return pl.pallas_call(
        _tc_body,
        grid=grid,
        in_specs=[
            pl.BlockSpec(memory_space=pltpu.SMEM),
            pl.BlockSpec((PHM, WBLK, WBLK), lambda i: (0, 0, 0)),
            pl.BlockSpec((ROW_BLK, HALF), lambda i: (i, 0)),
            pl.BlockSpec((ROW_BLK, HALF), lambda i: (i, 0)),
            pl.BlockSpec((ROW_BLK, D), lambda i: (i, 0)),
            pl.BlockSpec((1, D), lambda i: (0, 0)),
        ],
        out_specs=pl.BlockSpec((ROW_BLK, D), lambda i: (i, 0)),
        out_shape=jax.ShapeDtypeStruct((N, D), jnp.float32),
        scratch_shapes=[pltpu.VMEM((D, D), jnp.float32)],
    )(phm_rule, W, a0, a1, x, b2)


def kernel(x, edge_index, edge_attr, phm_rule, W, b):
    src = edge_index[0]
    dst = edge_index[1]
    x2 = x.reshape(2 * N, HALF)
    ea2 = edge_attr.reshape(2 * E, HALF)
    agg = _sc_aggregate(x2, ea2, src, dst)
    a0 = agg[:N]
    a1 = agg[N:]
    return _tc_phm(phm_rule, W, a0, a1, x, b.reshape(1, D))
